# baseline (speedup 1.0000x reference)
import functools

import jax
import jax.numpy as jnp
from jax import lax
from jax.experimental import pallas as pl
from jax.experimental.pallas import tpu as pltpu
from jax.experimental.pallas import tpu_sc as plsc

N = 10000
D_IN = 128
HID = 128
D_OUT = 64
K = 10
ALPHA = 0.1

NC = 2
NS = 16
NW = NC * NS
C = 128

NP = 10240
STRIPE = NP // NS
B = 1024
NBUF = 4
NB = NP // B

_mesh = plsc.VectorSubcoreMesh(
    core_axis_name="c", subcore_axis_name="s", num_cores=NC, num_subcores=NS)

_sc_params = pltpu.CompilerParams(use_tc_tiling_on_sc=False)


def _deg_body(cpt, cols_hbm, zeros_hbm, ones_hbm, degp_hbm,
              idxc, ones_v, acc, sem):
    c = lax.axis_index("c")
    s = lax.axis_index("s")
    wid = c * NS + s
    st = pl.ds(s * STRIPE, STRIPE)
    pltpu.sync_copy(cols_hbm.at[wid], idxc)
    pltpu.sync_copy(ones_hbm, ones_v)
    pltpu.sync_copy(zeros_hbm.at[st], acc.at[st])
    plsc.subcore_barrier()

    for j0 in range(3):
        pltpu.async_copy(ones_v, acc.at[idxc.at[j0]], sem, add=True)

    def body(j, carry):
        @pl.when(j + 3 < cpt)
        def _():
            pltpu.async_copy(ones_v, acc.at[idxc.at[j + 3]], sem, add=True)

        pltpu.make_async_copy(ones_v, acc.at[idxc.at[j]], sem).wait()
        return carry

    lax.fori_loop(0, cpt, body, 0)
    plsc.subcore_barrier()
    pltpu.sync_copy(acc.at[st], degp_hbm.at[c, st])


def _make_deg_kernel(cpt):
    return pl.kernel(
        functools.partial(_deg_body, cpt),
        out_type=jax.ShapeDtypeStruct((NC, NP), jnp.float32),
        mesh=_mesh,
        compiler_params=_sc_params,
        scratch_types=[
            pltpu.VMEM((cpt, C), jnp.int32),
            pltpu.VMEM((C,), jnp.float32),
            pltpu.VMEM_SHARED((NP,), jnp.float32),
            pltpu.SemaphoreType.DMA,
        ],
    )


def _mega_body(cpt, w0_hbm, hb_hbm, sm_hbm, rows_hbm, cols_hbm,
               p_hbm, wown_hbm, idxr, idxc, buf, acc, sem, bsem):
    c = lax.axis_index("c")
    s = lax.axis_index("s")
    wid = c * NS + s
    st = pl.ds(s * STRIPE, STRIPE)
    pltpu.sync_copy(rows_hbm.at[wid], idxr)
    pltpu.sync_copy(cols_hbm.at[wid], idxc)

    def xbarrier():
        plsc.subcore_barrier()

        @pl.when(s == 0)
        def _():
            pltpu.core_barrier(bsem, core_axis_name="c")

        plsc.subcore_barrier()

    pltpu.sync_copy(w0_hbm.at[st], wown_hbm.at[c, st])

    @pl.when(c == 0)
    def _():
        pltpu.sync_copy(w0_hbm.at[st], acc.at[st])

    @pl.when(c == 1)
    def _():
        pltpu.sync_copy(hb_hbm.at[st], acc.at[st])

    plsc.subcore_barrier()

    def step(k, carry):
        for j0 in range(NBUF - 1):
            pltpu.async_copy(wown_hbm.at[c].at[idxr.at[j0]], buf.at[j0], sem)

        def body(j, carry2):
            b = lax.rem(j, NBUF)
            fb = lax.rem(j + NBUF - 1, NBUF)

            @pl.when(j + NBUF - 1 < cpt)
            def _():
                pltpu.async_copy(
                    wown_hbm.at[c].at[idxr.at[j + NBUF - 1]], buf.at[fb], sem)

            pltpu.make_async_copy(
                wown_hbm.at[c].at[idxr.at[j]], buf.at[b], sem).wait()
            pltpu.sync_copy(buf.at[b], acc.at[idxc.at[j]], add=True)
            return carry2

        lax.fori_loop(0, cpt, body, 0)
        plsc.subcore_barrier()

        pltpu.sync_copy(acc.at[st], p_hbm.at[c, st])
        xbarrier()

        oc = 1 - c
        sel = jnp.where(k < K - 1, 0, 1)
        nq = STRIPE // C
        pltpu.async_copy(
            p_hbm.at[oc, pl.ds(s * STRIPE, C)], buf.at[0], sem)
        pltpu.async_copy(
            sm_hbm.at[sel, pl.ds(s * STRIPE, C)], buf.at[4], sem)
        for q in range(nq):
            qs = pl.ds(s * STRIPE + q * C, C)
            pb = q % 2
            sb = 4 + q % 2
            if q + 1 < nq:
                nxt = pl.ds(s * STRIPE + (q + 1) * C, C)
                pltpu.async_copy(p_hbm.at[oc, nxt], buf.at[(q + 1) % 2], sem)
                pltpu.async_copy(sm_hbm.at[sel, nxt], buf.at[4 + (q + 1) % 2], sem)
            pltpu.sync_copy(acc.at[qs], buf.at[2])
            pltpu.make_async_copy(p_hbm.at[oc, qs], buf.at[pb], sem).wait()
            pltpu.make_async_copy(sm_hbm.at[sel, qs], buf.at[sb], sem).wait()

            def cbody(r, carry3, _pb=pb, _sb=sb):
                for u in range(2):
                    for t in range(4):
                        ts = pl.ds(t * 16, 16)
                        buf[3, 2 * r + u, ts] = buf[_sb, 2 * r + u, ts] * (
                            buf[2, 2 * r + u, ts] + buf[_pb, 2 * r + u, ts])
                return carry3

            lax.fori_loop(0, C // 2, cbody, 0)
            pltpu.sync_copy(buf.at[3], wown_hbm.at[c, qs])

            @pl.when((c == 0) & (k < K - 1))
            def _():
                pltpu.sync_copy(buf.at[3], acc.at[qs])

        @pl.when((c == 1) & (k < K - 1))
        def _():
            pltpu.sync_copy(hb_hbm.at[st], acc.at[st])

        @pl.when(k < K - 1)
        def _():
            xbarrier()

        return carry

    lax.fori_loop(0, K, step, 0)


def _make_mega_kernel(cpt):
    return pl.kernel(
        functools.partial(_mega_body, cpt),
        out_type=(
            jax.ShapeDtypeStruct((NC, NP, D_OUT), jnp.float32),
            jax.ShapeDtypeStruct((NC, NP, D_OUT), jnp.float32),
        ),
        mesh=_mesh,
        compiler_params=_sc_params,
        scratch_types=[
            pltpu.VMEM((cpt, C), jnp.int32),
            pltpu.VMEM((cpt, C), jnp.int32),
            pltpu.VMEM((NBUF + 2, C, D_OUT), jnp.float32),
            pltpu.VMEM_SHARED((NP, D_OUT), jnp.float32),
            pltpu.SemaphoreType.DMA,
            pltpu.SemaphoreType.REGULAR,
        ],
    )


def _prep_block(x_ref, w1_ref, b1_ref, w2_ref, b2_ref, deg_ref,
                w0_ref, hb_ref, sm_ref, sf_ref):
    dr = deg_ref[...]
    deg = jnp.maximum(dr[0] + dr[1] + 1.0, 1.0)
    dis = lax.rsqrt(deg)
    dinv = 1.0 / deg
    h = jnp.dot(x_ref[...], w1_ref[...], preferred_element_type=jnp.float32)
    h = jnp.maximum(h + b1_ref[...], 0.0)
    h = jnp.dot(h, w2_ref[...], preferred_element_type=jnp.float32)
    h = h + b2_ref[...]
    w0_ref[...] = dis[:, None] * h
    hb_ref[...] = (ALPHA / (1.0 - ALPHA)) * (deg * dis)[:, None] * h
    sm_ref[...] = jnp.broadcast_to(((1.0 - ALPHA) * dinv)[:, None], (B, D_OUT))
    sf_ref[...] = jnp.broadcast_to(((1.0 - ALPHA) * dis)[:, None], (B, D_OUT))


_prep_kernel = pl.pallas_call(
    _prep_block,
    grid=(NB,),
    in_specs=[
        pl.BlockSpec((B, D_IN), lambda i: (i, 0)),
        pl.BlockSpec((D_IN, HID), lambda i: (0, 0)),
        pl.BlockSpec((1, HID), lambda i: (0, 0)),
        pl.BlockSpec((HID, D_OUT), lambda i: (0, 0)),
        pl.BlockSpec((1, D_OUT), lambda i: (0, 0)),
        pl.BlockSpec((NC, B), lambda i: (0, i)),
    ],
    out_specs=[
        pl.BlockSpec((B, D_OUT), lambda i: (i, 0)),
        pl.BlockSpec((B, D_OUT), lambda i: (i, 0)),
        pl.BlockSpec((B, D_OUT), lambda i: (i, 0)),
        pl.BlockSpec((B, D_OUT), lambda i: (i, 0)),
    ],
    out_shape=[
        jax.ShapeDtypeStruct((NP, D_OUT), jnp.float32),
        jax.ShapeDtypeStruct((NP, D_OUT), jnp.float32),
        jax.ShapeDtypeStruct((NP, D_OUT), jnp.float32),
        jax.ShapeDtypeStruct((NP, D_OUT), jnp.float32),
    ],
)


def _combine_block(p_ref, s_ref, w_ref):
    w_ref[...] = s_ref[...] * (p_ref[0] + p_ref[1])


_combine_kernel = pl.pallas_call(
    _combine_block,
    grid=(NB,),
    in_specs=[
        pl.BlockSpec((NC, B, D_OUT), lambda i: (0, i, 0)),
        pl.BlockSpec((B, 1), lambda i: (i, 0)),
    ],
    out_specs=pl.BlockSpec((B, D_OUT), lambda i: (i, 0)),
    out_shape=jax.ShapeDtypeStruct((NP, D_OUT), jnp.float32),
)


def kernel(x, edge_index, W1, b1, W2, b2):
    E = edge_index.shape[1]
    cpt = -(-E // (NW * C))
    epad = NW * cpt * C
    pad = epad - E

    row = edge_index[0]
    col = edge_index[1]
    if pad:
        pr = jnp.arange(pad, dtype=jnp.int32) % N
        pc = N + jnp.arange(pad, dtype=jnp.int32) % (NP - N)
        row = jnp.concatenate([row, pr])
        col = jnp.concatenate([col, pc])
    rows3 = row.reshape(NW, cpt, C)
    cols3 = col.reshape(NW, cpt, C)

    zeros_np = jnp.zeros((NP,), jnp.float32)
    ones_c = jnp.ones((C,), jnp.float32)
    degp = _make_deg_kernel(cpt)(cols3, zeros_np, ones_c)

    xp = jnp.pad(x, ((0, NP - N), (0, 0)))
    w0, hbv, smid, sfin = _prep_kernel(
        xp, W1, b1.reshape(1, HID), W2, b2.reshape(1, D_OUT), degp)
    scales = jnp.stack([smid, sfin])

    _, w = _make_mega_kernel(cpt)(w0, hbv, scales, rows3, cols3)
    return w[0, :N]


# speedup vs baseline: 1.0106x; 1.0106x over previous
import functools

import jax
import jax.numpy as jnp
from jax import lax
from jax.experimental import pallas as pl
from jax.experimental.pallas import tpu as pltpu
from jax.experimental.pallas import tpu_sc as plsc

N = 10000
D_IN = 128
HID = 128
D_OUT = 64
K = 10
ALPHA = 0.1

NC = 2
NS = 16
NW = NC * NS
C = 128

NP = 10240
STRIPE = NP // NS
B = 1024
NBUF = 4
NB = NP // B

_mesh = plsc.VectorSubcoreMesh(
    core_axis_name="c", subcore_axis_name="s", num_cores=NC, num_subcores=NS)

_sc_params = pltpu.CompilerParams(use_tc_tiling_on_sc=False)


def _deg_body(cpt, cols_hbm, zeros_hbm, ones_hbm, degp_hbm,
              idxc, ones_v, acc, sem):
    c = lax.axis_index("c")
    s = lax.axis_index("s")
    wid = c * NS + s
    st = pl.ds(s * STRIPE, STRIPE)
    pltpu.sync_copy(cols_hbm.at[wid], idxc)
    pltpu.sync_copy(ones_hbm, ones_v)
    pltpu.sync_copy(zeros_hbm.at[st], acc.at[st])
    plsc.subcore_barrier()

    for j0 in range(3):
        pltpu.async_copy(ones_v, acc.at[idxc.at[j0]], sem, add=True)

    def body(j, carry):
        @pl.when(j + 3 < cpt)
        def _():
            pltpu.async_copy(ones_v, acc.at[idxc.at[j + 3]], sem, add=True)

        pltpu.make_async_copy(ones_v, acc.at[idxc.at[j]], sem).wait()
        return carry

    lax.fori_loop(0, cpt, body, 0)
    plsc.subcore_barrier()
    pltpu.sync_copy(acc.at[st], degp_hbm.at[c, st])


def _make_deg_kernel(cpt):
    return pl.kernel(
        functools.partial(_deg_body, cpt),
        out_type=jax.ShapeDtypeStruct((NC, NP), jnp.float32),
        mesh=_mesh,
        compiler_params=_sc_params,
        scratch_types=[
            pltpu.VMEM((cpt, C), jnp.int32),
            pltpu.VMEM((C,), jnp.float32),
            pltpu.VMEM_SHARED((NP,), jnp.float32),
            pltpu.SemaphoreType.DMA,
        ],
    )


def _mega_body(cpt, w0_hbm, hb_hbm, sm_hbm, rows_hbm, cols_hbm,
               p_hbm, wown_hbm, idxr, idxc, buf, acc, sem, bsem):
    c = lax.axis_index("c")
    s = lax.axis_index("s")
    wid = c * NS + s
    st = pl.ds(s * STRIPE, STRIPE)
    pltpu.sync_copy(rows_hbm.at[wid], idxr)
    pltpu.sync_copy(cols_hbm.at[wid], idxc)

    def xbarrier():
        plsc.subcore_barrier()

        @pl.when(s == 0)
        def _():
            pltpu.core_barrier(bsem, core_axis_name="c")

        plsc.subcore_barrier()

    pltpu.sync_copy(w0_hbm.at[st], wown_hbm.at[c, st])

    @pl.when(c == 0)
    def _():
        pltpu.sync_copy(w0_hbm.at[st], acc.at[st])

    @pl.when(c == 1)
    def _():
        pltpu.sync_copy(hb_hbm.at[st], acc.at[st])

    plsc.subcore_barrier()

    def step(k, carry):
        for j0 in range(NBUF - 1):
            pltpu.async_copy(wown_hbm.at[c].at[idxr.at[j0]], buf.at[j0], sem)

        def body(j, carry2):
            b = lax.rem(j, NBUF)
            fb = lax.rem(j + NBUF - 1, NBUF)

            @pl.when(j + NBUF - 1 < cpt)
            def _():
                pltpu.async_copy(
                    wown_hbm.at[c].at[idxr.at[j + NBUF - 1]], buf.at[fb], sem)

            pltpu.make_async_copy(
                wown_hbm.at[c].at[idxr.at[j]], buf.at[b], sem).wait()
            pltpu.sync_copy(buf.at[b], acc.at[idxc.at[j]], add=True)
            return carry2

        lax.fori_loop(0, cpt, body, 0)
        plsc.subcore_barrier()

        pltpu.sync_copy(acc.at[st], p_hbm.at[c, st])
        xbarrier()

        @pl.when(k < K - 1)
        def _():
            _combine_phase(k)

        return carry

    def _combine_phase(k):
        oc = 1 - c
        sel = 0
        nq = STRIPE // C
        pltpu.async_copy(
            p_hbm.at[oc, pl.ds(s * STRIPE, C)], buf.at[0], sem)
        pltpu.async_copy(
            sm_hbm.at[sel, pl.ds(s * STRIPE, C)], buf.at[4], sem)
        for q in range(nq):
            qs = pl.ds(s * STRIPE + q * C, C)
            pb = q % 2
            sb = 4 + q % 2
            if q + 1 < nq:
                nxt = pl.ds(s * STRIPE + (q + 1) * C, C)
                pltpu.async_copy(p_hbm.at[oc, nxt], buf.at[(q + 1) % 2], sem)
                pltpu.async_copy(sm_hbm.at[sel, nxt], buf.at[4 + (q + 1) % 2], sem)
            pltpu.sync_copy(acc.at[qs], buf.at[2])
            pltpu.make_async_copy(p_hbm.at[oc, qs], buf.at[pb], sem).wait()
            pltpu.make_async_copy(sm_hbm.at[sel, qs], buf.at[sb], sem).wait()

            def cbody(r, carry3, _pb=pb, _sb=sb):
                for u in range(2):
                    for t in range(4):
                        ts = pl.ds(t * 16, 16)
                        buf[3, 2 * r + u, ts] = buf[_sb, 2 * r + u, ts] * (
                            buf[2, 2 * r + u, ts] + buf[_pb, 2 * r + u, ts])
                return carry3

            lax.fori_loop(0, C // 2, cbody, 0)
            pltpu.sync_copy(buf.at[3], wown_hbm.at[c, qs])

            @pl.when(c == 0)
            def _():
                pltpu.sync_copy(buf.at[3], acc.at[qs])

        @pl.when(c == 1)
        def _():
            pltpu.sync_copy(hb_hbm.at[st], acc.at[st])

        xbarrier()

    lax.fori_loop(0, K, step, 0)


def _make_mega_kernel(cpt):
    return pl.kernel(
        functools.partial(_mega_body, cpt),
        out_type=(
            jax.ShapeDtypeStruct((NC, NP, D_OUT), jnp.float32),
            jax.ShapeDtypeStruct((NC, NP, D_OUT), jnp.float32),
        ),
        mesh=_mesh,
        compiler_params=_sc_params,
        scratch_types=[
            pltpu.VMEM((cpt, C), jnp.int32),
            pltpu.VMEM((cpt, C), jnp.int32),
            pltpu.VMEM((NBUF + 2, C, D_OUT), jnp.float32),
            pltpu.VMEM_SHARED((NP, D_OUT), jnp.float32),
            pltpu.SemaphoreType.DMA,
            pltpu.SemaphoreType.REGULAR,
        ],
    )


def _prep_block(x_ref, w1_ref, b1_ref, w2_ref, b2_ref, deg_ref,
                w0_ref, hb_ref, sm_ref, sf_ref):
    dr = deg_ref[...]
    deg = jnp.maximum(dr[0] + dr[1] + 1.0, 1.0)
    dis = lax.rsqrt(deg)
    dinv = 1.0 / deg
    h = jnp.dot(x_ref[...], w1_ref[...], preferred_element_type=jnp.float32)
    h = jnp.maximum(h + b1_ref[...], 0.0)
    h = jnp.dot(h, w2_ref[...], preferred_element_type=jnp.float32)
    h = h + b2_ref[...]
    w0_ref[...] = dis[:, None] * h
    hb_ref[...] = (ALPHA / (1.0 - ALPHA)) * (deg * dis)[:, None] * h
    sm_ref[...] = jnp.broadcast_to(((1.0 - ALPHA) * dinv)[:, None], (B, D_OUT))
    sf_ref[...] = jnp.broadcast_to(((1.0 - ALPHA) * dis)[:, None], (B, D_OUT))


_prep_kernel = pl.pallas_call(
    _prep_block,
    grid=(NB,),
    in_specs=[
        pl.BlockSpec((B, D_IN), lambda i: (i, 0)),
        pl.BlockSpec((D_IN, HID), lambda i: (0, 0)),
        pl.BlockSpec((1, HID), lambda i: (0, 0)),
        pl.BlockSpec((HID, D_OUT), lambda i: (0, 0)),
        pl.BlockSpec((1, D_OUT), lambda i: (0, 0)),
        pl.BlockSpec((NC, B), lambda i: (0, i)),
    ],
    out_specs=[
        pl.BlockSpec((B, D_OUT), lambda i: (i, 0)),
        pl.BlockSpec((B, D_OUT), lambda i: (i, 0)),
        pl.BlockSpec((B, D_OUT), lambda i: (i, 0)),
        pl.BlockSpec((B, D_OUT), lambda i: (i, 0)),
    ],
    out_shape=[
        jax.ShapeDtypeStruct((NP, D_OUT), jnp.float32),
        jax.ShapeDtypeStruct((NP, D_OUT), jnp.float32),
        jax.ShapeDtypeStruct((NP, D_OUT), jnp.float32),
        jax.ShapeDtypeStruct((NP, D_OUT), jnp.float32),
    ],
)


def _combine_block(p_ref, s_ref, w_ref):
    w_ref[...] = s_ref[...] * (p_ref[0] + p_ref[1])


_combine_kernel = pl.pallas_call(
    _combine_block,
    grid=(NB,),
    in_specs=[
        pl.BlockSpec((NC, B, D_OUT), lambda i: (0, i, 0)),
        pl.BlockSpec((B, 1), lambda i: (i, 0)),
    ],
    out_specs=pl.BlockSpec((B, D_OUT), lambda i: (i, 0)),
    out_shape=jax.ShapeDtypeStruct((NP, D_OUT), jnp.float32),
)


def kernel(x, edge_index, W1, b1, W2, b2):
    E = edge_index.shape[1]
    cpt = -(-E // (NW * C))
    epad = NW * cpt * C
    pad = epad - E

    row = edge_index[0]
    col = edge_index[1]
    if pad:
        pr = jnp.arange(pad, dtype=jnp.int32) % N
        pc = N + jnp.arange(pad, dtype=jnp.int32) % (NP - N)
        row = jnp.concatenate([row, pr])
        col = jnp.concatenate([col, pc])
    rows3 = row.reshape(NW, cpt, C)
    cols3 = col.reshape(NW, cpt, C)

    zeros_np = jnp.zeros((NP,), jnp.float32)
    ones_c = jnp.ones((C,), jnp.float32)
    degp = _make_deg_kernel(cpt)(cols3, zeros_np, ones_c)

    xp = jnp.pad(x, ((0, NP - N), (0, 0)))
    w0, hbv, smid, sfin = _prep_kernel(
        xp, W1, b1.reshape(1, HID), W2, b2.reshape(1, D_OUT), degp)
    scales = smid.reshape(1, NP, D_OUT)

    p, _ = _make_mega_kernel(cpt)(w0, hbv, scales, rows3, cols3)
    w = _combine_kernel(p, sfin[:, :1])
    return w[:N]
